# Initial kernel scaffold; baseline (speedup 1.0000x reference)
import functools
import jax
import jax.numpy as jnp
from jax import lax
from jax.experimental import pallas as pl
from jax.experimental.pallas import tpu as pltpu
from jax.experimental.pallas import tpu_sc as plsc

_B = 2
_N = 10000
_F = 128
_E = 160000
_C = 10
_NCLS = 2
_THR = 0.1

_NTILES = 32
_CHUNK = 128
_NCHUNK = 40
_E_PAD = _NTILES * _NCHUNK * _CHUNK
_ROWS_PER_TILE = 626
_N_PAD = 16 * _ROWS_PER_TILE

_RBLK = 1000
_NBLK = _N // _RBLK


def _mm_body(x_ref, w1_ref, w2_ref, y1_ref, y2_ref):
    xb = x_ref[...]
    y1_ref[...] = jnp.dot(xb, w1_ref[...], preferred_element_type=jnp.float32)
    y2_ref[...] = jnp.dot(xb, w2_ref[...], preferred_element_type=jnp.float32)


def _softmax_thr(lg):
    m = jnp.max(lg, axis=-1, keepdims=True)
    e = jnp.exp(lg - m)
    s = e / jnp.sum(e, axis=-1, keepdims=True)
    return jnp.where(s >= _THR, s, jnp.zeros_like(s))


def _post_body(x_ref, ys_ref, a00_ref, a01_ref, a10_ref, a11_ref,
               bph_ref, bnc_ref,
               snb_ref, sph_sum_ref, snb_sum_ref, xp_ref):
    i = pl.program_id(0)

    @pl.when(i == 0)
    def _():
        sph_sum_ref[...] = jnp.zeros_like(sph_sum_ref)
        snb_sum_ref[...] = jnp.zeros_like(snb_sum_ref)
        xp_ref[...] = jnp.zeros_like(xp_ref)

    aggs = (a00_ref[...] + a01_ref[...], a10_ref[...] + a11_ref[...])
    for b in range(_B):
        agg = aggs[b]
        ys = ys_ref[b]
        s_ph = _softmax_thr(ys[:, :_C] + agg[:, :_C] + bph_ref[...])
        s_nb = _softmax_thr(ys[:, _C:2 * _C] + agg[:, _C:2 * _C] + bnc_ref[...])
        snb_ref[b] = s_nb
        sph_sum_ref[b] += jnp.sum(s_ph, axis=0)
        snb_sum_ref[b] += jnp.sum(s_nb, axis=0)
        xp_ref[b] += lax.dot_general(
            s_nb, x_ref[b], (((0,), (0,)), ((), ())),
            preferred_element_type=jnp.float32)


def _fin_body(t00_ref, t01_ref, t10_ref, t11_ref, snb_ref,
              sph_sum_ref, snb_sum_ref, xp_ref,
              wacs_ref, wacn_ref, bac_ref, wlin_ref, blin_ref,
              ap_ref, out_ref):
    i = pl.program_id(0)

    @pl.when(i == 0)
    def _():
        ap_ref[...] = jnp.zeros_like(ap_ref)

    ts = (t00_ref[...] + t01_ref[...], t10_ref[...] + t11_ref[...])
    for b in range(_B):
        t = ts[b][:, :_C]
        ap_ref[b] += lax.dot_general(
            t, snb_ref[b], (((0,), (0,)), ((), ())),
            preferred_element_type=jnp.float32)

    @pl.when(i == _NBLK - 1)
    def _():
        rows = []
        inv_n = 1.0 / _N
        for b in range(_B):
            s_ph_m = sph_sum_ref[b].reshape(1, _C) * inv_n
            s_nb_m = snb_sum_ref[b].reshape(1, _C) * inv_n
            a_p = ap_ref[b]
            x_p = xp_ref[b]
            agg_a = jnp.dot(a_p, x_p, preferred_element_type=jnp.float32)
            s_ar = (jnp.dot(x_p, wacs_ref[...], preferred_element_type=jnp.float32)
                    + jnp.dot(agg_a, wacn_ref[...], preferred_element_type=jnp.float32)
                    + bac_ref[...])
            s_ar = _softmax_thr(s_ar)
            s_ar_m = jnp.sum(s_ar, axis=0, keepdims=True) * (1.0 / _C)
            scat = jnp.concatenate([s_ph_m, s_nb_m, s_ar_m], axis=-1)
            rows.append(jnp.dot(scat, wlin_ref[...],
                                preferred_element_type=jnp.float32) + blin_ref[...])
        out_ref[...] = jnp.concatenate(rows, axis=0)


def _make_segsum(ch):
    mesh = plsc.VectorSubcoreMesh(core_axis_name="c", subcore_axis_name="s")
    out_sds = jax.ShapeDtypeStruct((2, _N_PAD, ch), jnp.float32)

    @functools.partial(
        pl.kernel,
        out_type=(out_sds, out_sds),
        mesh=mesh,
        scratch_types=[
            pltpu.VMEM((_NCHUNK, _CHUNK), jnp.int32),
            pltpu.VMEM((_NCHUNK, _CHUNK), jnp.int32),
            pltpu.VMEM((_CHUNK, ch), jnp.float32),
            pltpu.VMEM_SHARED((_N_PAD, ch), jnp.float32),
            pltpu.VMEM_SHARED((_N_PAD, ch), jnp.float32),
            pltpu.SemaphoreType.DMA,
            pltpu.SemaphoreType.DMA,
        ],
    )
    def segsum(t0_hbm, t1_hbm, srcc_hbm, dstc_hbm, zero_hbm,
               out0_hbm, out1_hbm,
               src_v, dst_v, rows_v, acc0, acc1, gsem, ssem):
        c = lax.axis_index("c")
        s = lax.axis_index("s")
        tid = c * 16 + s
        rbase = s * _ROWS_PER_TILE

        pltpu.sync_copy(zero_hbm.at[pl.ds(rbase, _ROWS_PER_TILE)],
                        acc0.at[pl.ds(rbase, _ROWS_PER_TILE)])
        pltpu.sync_copy(zero_hbm.at[pl.ds(rbase, _ROWS_PER_TILE)],
                        acc1.at[pl.ds(rbase, _ROWS_PER_TILE)])
        pltpu.sync_copy(srcc_hbm.at[tid], src_v)
        pltpu.sync_copy(dstc_hbm.at[tid], dst_v)
        plsc.subcore_barrier()

        for tbl, acc in ((t0_hbm, acc0), (t1_hbm, acc1)):
            def chunk(j, carry, tbl=tbl, acc=acc):
                pltpu.async_copy(tbl.at[src_v.at[j]], rows_v, gsem).wait()
                pltpu.async_copy(rows_v, acc.at[dst_v.at[j]], ssem,
                                 add=True).wait()
                return carry
            lax.fori_loop(0, _NCHUNK, chunk, 0)

        plsc.subcore_barrier()
        for acc, out in ((acc0, out0_hbm), (acc1, out1_hbm)):
            pltpu.sync_copy(acc.at[pl.ds(rbase, _ROWS_PER_TILE)],
                            out.at[c].at[pl.ds(rbase, _ROWS_PER_TILE)])

    return segsum


_segsum32 = _make_segsum(32)
_segsum16 = _make_segsum(16)


def kernel(x, edge_index, W_ph_self, W_ph_nb, b_ph, W_nc_self, W_nc_nb, b_nc,
           W_ac_self, W_ac_nb, b_ac, W_lin, b_lin):
    f32 = jnp.float32
    src = edge_index[0]
    dst = edge_index[1]
    pad = jnp.full((_E_PAD - _E,), _N, dtype=jnp.int32)
    srcc = jnp.concatenate([src, pad]).reshape(_NTILES, _NCHUNK, _CHUNK)
    dstc = jnp.concatenate([dst, pad]).reshape(_NTILES, _NCHUNK, _CHUNK)

    w1 = jnp.concatenate([W_ph_self, W_nc_self], axis=1)
    w2 = jnp.concatenate([W_ph_nb, W_nc_nb,
                          jnp.zeros((_F, 12), f32)], axis=1)
    x2 = x.reshape(_B * _N, _F)
    ys, ynb = pl.pallas_call(
        _mm_body,
        grid=(_B * _NBLK,),
        in_specs=[
            pl.BlockSpec((_RBLK, _F), lambda i: (i, 0)),
            pl.BlockSpec((_F, 2 * _C), lambda i: (0, 0)),
            pl.BlockSpec((_F, 32), lambda i: (0, 0)),
        ],
        out_specs=[
            pl.BlockSpec((_RBLK, 2 * _C), lambda i: (i, 0)),
            pl.BlockSpec((_RBLK, 32), lambda i: (i, 0)),
        ],
        out_shape=[jax.ShapeDtypeStruct((_B * _N, 2 * _C), f32),
                   jax.ShapeDtypeStruct((_B * _N, 32), f32)],
    )(x2, w1, w2)

    ynb3 = jnp.pad(ynb.reshape(_B, _N, 32), ((0, 0), (0, _N_PAD - _N), (0, 0)))
    zero32 = jnp.zeros((_N_PAD, 32), f32)
    agg0, agg1 = _segsum32(ynb3[0], ynb3[1], srcc, dstc, zero32)

    ys3 = ys.reshape(_B, _N, 2 * _C)
    snb, sph_sum, snb_sum, xp = pl.pallas_call(
        _post_body,
        grid=(_NBLK,),
        in_specs=[
            pl.BlockSpec((_B, _RBLK, _F), lambda i: (0, i, 0)),
            pl.BlockSpec((_B, _RBLK, 2 * _C), lambda i: (0, i, 0)),
            pl.BlockSpec((_RBLK, 32), lambda i: (i, 0)),
            pl.BlockSpec((_RBLK, 32), lambda i: (i, 0)),
            pl.BlockSpec((_RBLK, 32), lambda i: (i, 0)),
            pl.BlockSpec((_RBLK, 32), lambda i: (i, 0)),
            pl.BlockSpec((1, _C), lambda i: (0, 0)),
            pl.BlockSpec((1, _C), lambda i: (0, 0)),
        ],
        out_specs=[
            pl.BlockSpec((_B, _RBLK, _C), lambda i: (0, i, 0)),
            pl.BlockSpec((_B, _C), lambda i: (0, 0)),
            pl.BlockSpec((_B, _C), lambda i: (0, 0)),
            pl.BlockSpec((_B, _C, _F), lambda i: (0, 0, 0)),
        ],
        out_shape=[jax.ShapeDtypeStruct((_B, _N, _C), f32),
                   jax.ShapeDtypeStruct((_B, _C), f32),
                   jax.ShapeDtypeStruct((_B, _C), f32),
                   jax.ShapeDtypeStruct((_B, _C, _F), f32)],
    )(x, ys3, agg0[0, :_N], agg0[1, :_N], agg1[0, :_N], agg1[1, :_N],
      b_ph.reshape(1, _C), b_nc.reshape(1, _C))

    snb16 = jnp.pad(snb, ((0, 0), (0, _N_PAD - _N), (0, 16 - _C)))
    zero16 = jnp.zeros((_N_PAD, 16), f32)
    t0, t1 = _segsum16(snb16[0], snb16[1], srcc, dstc, zero16)

    _, logits = pl.pallas_call(
        _fin_body,
        grid=(_NBLK,),
        in_specs=[
            pl.BlockSpec((_RBLK, 16), lambda i: (i, 0)),
            pl.BlockSpec((_RBLK, 16), lambda i: (i, 0)),
            pl.BlockSpec((_RBLK, 16), lambda i: (i, 0)),
            pl.BlockSpec((_RBLK, 16), lambda i: (i, 0)),
            pl.BlockSpec((_B, _RBLK, _C), lambda i: (0, i, 0)),
            pl.BlockSpec((_B, _C), lambda i: (0, 0)),
            pl.BlockSpec((_B, _C), lambda i: (0, 0)),
            pl.BlockSpec((_B, _C, _F), lambda i: (0, 0, 0)),
            pl.BlockSpec((_F, _C), lambda i: (0, 0)),
            pl.BlockSpec((_F, _C), lambda i: (0, 0)),
            pl.BlockSpec((1, _C), lambda i: (0, 0)),
            pl.BlockSpec((3 * _C, _NCLS), lambda i: (0, 0)),
            pl.BlockSpec((1, _NCLS), lambda i: (0, 0)),
        ],
        out_specs=[
            pl.BlockSpec((_B, _C, _C), lambda i: (0, 0, 0)),
            pl.BlockSpec((_B, _NCLS), lambda i: (0, 0)),
        ],
        out_shape=[jax.ShapeDtypeStruct((_B, _C, _C), f32),
                   jax.ShapeDtypeStruct((_B, _NCLS), f32)],
    )(t0[0, :_N], t0[1, :_N], t1[0, :_N], t1[1, :_N], snb,
      sph_sum, snb_sum, xp,
      W_ac_self, W_ac_nb, b_ac.reshape(1, _C),
      W_lin, b_lin.reshape(1, _NCLS))

    return logits


# speedup vs baseline: 39.9890x; 39.9890x over previous
import functools
import jax
import jax.numpy as jnp
from jax import lax
from jax.experimental import pallas as pl
from jax.experimental.pallas import tpu as pltpu
from jax.experimental.pallas import tpu_sc as plsc

_B = 2
_N = 10000
_F = 128
_E = 160000
_C = 10
_NCLS = 2
_THR = 0.1

_NTILES = 32
_CHUNK = 128
_NCHUNK = 40
_E_PAD = _NTILES * _NCHUNK * _CHUNK
_ROWS_PER_TILE = 632
_N_PAD = 16 * _ROWS_PER_TILE

_RBLK = 1000
_NBLK = _N // _RBLK


def _mm_body(x_ref, w1_ref, w2_ref, y1_ref, y2_ref):
    xb = x_ref[...]
    y1_ref[...] = jnp.dot(xb, w1_ref[...], preferred_element_type=jnp.float32)
    y2_ref[...] = jnp.dot(xb, w2_ref[...], preferred_element_type=jnp.float32)


def _softmax_thr(lg):
    m = jnp.max(lg, axis=-1, keepdims=True)
    e = jnp.exp(lg - m)
    s = e / jnp.sum(e, axis=-1, keepdims=True)
    return jnp.where(s >= _THR, s, jnp.zeros_like(s))


def _post_body(x_ref, ys_ref, a00_ref, a01_ref, a10_ref, a11_ref,
               bph_ref, bnc_ref,
               snb_ref, sph_sum_ref, snb_sum_ref, xp_ref):
    i = pl.program_id(0)

    @pl.when(i == 0)
    def _():
        sph_sum_ref[...] = jnp.zeros_like(sph_sum_ref)
        snb_sum_ref[...] = jnp.zeros_like(snb_sum_ref)
        xp_ref[...] = jnp.zeros_like(xp_ref)

    aggs = (a00_ref[...] + a01_ref[...], a10_ref[...] + a11_ref[...])
    for b in range(_B):
        agg = aggs[b]
        ys = ys_ref[b]
        s_ph = _softmax_thr(ys[:, :_C] + agg[:, :_C] + bph_ref[...])
        s_nb = _softmax_thr(ys[:, _C:2 * _C] + agg[:, _C:2 * _C] + bnc_ref[...])
        snb_ref[b] = s_nb
        sph_sum_ref[b] += jnp.sum(s_ph, axis=0)
        snb_sum_ref[b] += jnp.sum(s_nb, axis=0)
        xp_ref[b] += lax.dot_general(
            s_nb, x_ref[b], (((0,), (0,)), ((), ())),
            preferred_element_type=jnp.float32)


def _fin_body(t00_ref, t01_ref, t10_ref, t11_ref, snb_ref,
              sph_sum_ref, snb_sum_ref, xp_ref,
              wacs_ref, wacn_ref, bac_ref, wlin_ref, blin_ref,
              ap_ref, out_ref):
    i = pl.program_id(0)

    @pl.when(i == 0)
    def _():
        ap_ref[...] = jnp.zeros_like(ap_ref)

    ts = (t00_ref[...] + t01_ref[...], t10_ref[...] + t11_ref[...])
    for b in range(_B):
        t = ts[b][:, :_C]
        ap_ref[b] += lax.dot_general(
            t, snb_ref[b], (((0,), (0,)), ((), ())),
            preferred_element_type=jnp.float32)

    @pl.when(i == _NBLK - 1)
    def _():
        rows = []
        inv_n = 1.0 / _N
        for b in range(_B):
            s_ph_m = sph_sum_ref[b].reshape(1, _C) * inv_n
            s_nb_m = snb_sum_ref[b].reshape(1, _C) * inv_n
            a_p = ap_ref[b]
            x_p = xp_ref[b]
            agg_a = jnp.dot(a_p, x_p, preferred_element_type=jnp.float32)
            s_ar = (jnp.dot(x_p, wacs_ref[...], preferred_element_type=jnp.float32)
                    + jnp.dot(agg_a, wacn_ref[...], preferred_element_type=jnp.float32)
                    + bac_ref[...])
            s_ar = _softmax_thr(s_ar)
            s_ar_m = jnp.sum(s_ar, axis=0, keepdims=True) * (1.0 / _C)
            scat = jnp.concatenate([s_ph_m, s_nb_m, s_ar_m], axis=-1)
            rows.append(jnp.dot(scat, wlin_ref[...],
                                preferred_element_type=jnp.float32) + blin_ref[...])
        out_ref[...] = jnp.concatenate(rows, axis=0)


def _make_segsum(ch):
    mesh = plsc.VectorSubcoreMesh(core_axis_name="c", subcore_axis_name="s")
    out_sds = jax.ShapeDtypeStruct((2, _N_PAD, ch), jnp.float32)

    @functools.partial(
        pl.kernel,
        out_type=(out_sds, out_sds),
        mesh=mesh,
        scratch_types=[
            pltpu.VMEM((_NCHUNK, _CHUNK), jnp.int32),
            pltpu.VMEM((_NCHUNK, _CHUNK), jnp.int32),
            pltpu.VMEM((_CHUNK, ch), jnp.float32),
            pltpu.VMEM_SHARED((_N_PAD, ch), jnp.float32),
            pltpu.VMEM_SHARED((_N_PAD, ch), jnp.float32),
            pltpu.SemaphoreType.DMA,
            pltpu.SemaphoreType.DMA,
        ],
        compiler_params=pltpu.CompilerParams(use_tc_tiling_on_sc=False),
    )
    def segsum(t0_hbm, t1_hbm, srcc_hbm, dstc_hbm, zero_hbm,
               out0_hbm, out1_hbm,
               src_v, dst_v, rows_v, acc0, acc1, gsem, ssem):
        c = lax.axis_index("c")
        s = lax.axis_index("s")
        tid = c * 16 + s
        rbase = s * _ROWS_PER_TILE

        pltpu.sync_copy(zero_hbm.at[pl.ds(rbase, _ROWS_PER_TILE)],
                        acc0.at[pl.ds(rbase, _ROWS_PER_TILE)])
        pltpu.sync_copy(zero_hbm.at[pl.ds(rbase, _ROWS_PER_TILE)],
                        acc1.at[pl.ds(rbase, _ROWS_PER_TILE)])
        pltpu.sync_copy(srcc_hbm.at[tid], src_v)
        pltpu.sync_copy(dstc_hbm.at[tid], dst_v)
        plsc.subcore_barrier()

        for tbl, acc in ((t0_hbm, acc0), (t1_hbm, acc1)):
            def chunk(j, carry, tbl=tbl, acc=acc):
                pltpu.async_copy(tbl.at[src_v.at[j]], rows_v, gsem).wait()
                pltpu.async_copy(rows_v, acc.at[dst_v.at[j]], ssem,
                                 add=True).wait()
                return carry
            lax.fori_loop(0, _NCHUNK, chunk, 0)

        plsc.subcore_barrier()
        for acc, out in ((acc0, out0_hbm), (acc1, out1_hbm)):
            pltpu.sync_copy(acc.at[pl.ds(rbase, _ROWS_PER_TILE)],
                            out.at[c].at[pl.ds(rbase, _ROWS_PER_TILE)])

    return segsum


_segsum32 = _make_segsum(32)
_segsum16 = _make_segsum(16)


def kernel(x, edge_index, W_ph_self, W_ph_nb, b_ph, W_nc_self, W_nc_nb, b_nc,
           W_ac_self, W_ac_nb, b_ac, W_lin, b_lin):
    f32 = jnp.float32
    src = edge_index[0]
    dst = edge_index[1]
    pad = jnp.full((_E_PAD - _E,), _N, dtype=jnp.int32)
    srcc = jnp.concatenate([src, pad]).reshape(_NTILES, _NCHUNK, _CHUNK)
    dstc = jnp.concatenate([dst, pad]).reshape(_NTILES, _NCHUNK, _CHUNK)

    w1 = jnp.concatenate([W_ph_self, W_nc_self], axis=1)
    w2 = jnp.concatenate([W_ph_nb, W_nc_nb,
                          jnp.zeros((_F, 12), f32)], axis=1)
    x2 = x.reshape(_B * _N, _F)
    ys, ynb = pl.pallas_call(
        _mm_body,
        grid=(_B * _NBLK,),
        in_specs=[
            pl.BlockSpec((_RBLK, _F), lambda i: (i, 0)),
            pl.BlockSpec((_F, 2 * _C), lambda i: (0, 0)),
            pl.BlockSpec((_F, 32), lambda i: (0, 0)),
        ],
        out_specs=[
            pl.BlockSpec((_RBLK, 2 * _C), lambda i: (i, 0)),
            pl.BlockSpec((_RBLK, 32), lambda i: (i, 0)),
        ],
        out_shape=[jax.ShapeDtypeStruct((_B * _N, 2 * _C), f32),
                   jax.ShapeDtypeStruct((_B * _N, 32), f32)],
    )(x2, w1, w2)

    ynb3 = jnp.pad(ynb.reshape(_B, _N, 32), ((0, 0), (0, _N_PAD - _N), (0, 0)))
    zero32 = jnp.zeros((_N_PAD, 32), f32)
    agg0, agg1 = _segsum32(ynb3[0], ynb3[1], srcc, dstc, zero32)

    ys3 = ys.reshape(_B, _N, 2 * _C)
    snb, sph_sum, snb_sum, xp = pl.pallas_call(
        _post_body,
        grid=(_NBLK,),
        in_specs=[
            pl.BlockSpec((_B, _RBLK, _F), lambda i: (0, i, 0)),
            pl.BlockSpec((_B, _RBLK, 2 * _C), lambda i: (0, i, 0)),
            pl.BlockSpec((_RBLK, 32), lambda i: (i, 0)),
            pl.BlockSpec((_RBLK, 32), lambda i: (i, 0)),
            pl.BlockSpec((_RBLK, 32), lambda i: (i, 0)),
            pl.BlockSpec((_RBLK, 32), lambda i: (i, 0)),
            pl.BlockSpec((1, _C), lambda i: (0, 0)),
            pl.BlockSpec((1, _C), lambda i: (0, 0)),
        ],
        out_specs=[
            pl.BlockSpec((_B, _RBLK, _C), lambda i: (0, i, 0)),
            pl.BlockSpec((_B, _C), lambda i: (0, 0)),
            pl.BlockSpec((_B, _C), lambda i: (0, 0)),
            pl.BlockSpec((_B, _C, _F), lambda i: (0, 0, 0)),
        ],
        out_shape=[jax.ShapeDtypeStruct((_B, _N, _C), f32),
                   jax.ShapeDtypeStruct((_B, _C), f32),
                   jax.ShapeDtypeStruct((_B, _C), f32),
                   jax.ShapeDtypeStruct((_B, _C, _F), f32)],
    )(x, ys3, agg0[0, :_N], agg0[1, :_N], agg1[0, :_N], agg1[1, :_N],
      b_ph.reshape(1, _C), b_nc.reshape(1, _C))

    snb16 = jnp.pad(snb, ((0, 0), (0, _N_PAD - _N), (0, 16 - _C)))
    zero16 = jnp.zeros((_N_PAD, 16), f32)
    t0, t1 = _segsum16(snb16[0], snb16[1], srcc, dstc, zero16)

    _, logits = pl.pallas_call(
        _fin_body,
        grid=(_NBLK,),
        in_specs=[
            pl.BlockSpec((_RBLK, 16), lambda i: (i, 0)),
            pl.BlockSpec((_RBLK, 16), lambda i: (i, 0)),
            pl.BlockSpec((_RBLK, 16), lambda i: (i, 0)),
            pl.BlockSpec((_RBLK, 16), lambda i: (i, 0)),
            pl.BlockSpec((_B, _RBLK, _C), lambda i: (0, i, 0)),
            pl.BlockSpec((_B, _C), lambda i: (0, 0)),
            pl.BlockSpec((_B, _C), lambda i: (0, 0)),
            pl.BlockSpec((_B, _C, _F), lambda i: (0, 0, 0)),
            pl.BlockSpec((_F, _C), lambda i: (0, 0)),
            pl.BlockSpec((_F, _C), lambda i: (0, 0)),
            pl.BlockSpec((1, _C), lambda i: (0, 0)),
            pl.BlockSpec((3 * _C, _NCLS), lambda i: (0, 0)),
            pl.BlockSpec((1, _NCLS), lambda i: (0, 0)),
        ],
        out_specs=[
            pl.BlockSpec((_B, _C, _C), lambda i: (0, 0, 0)),
            pl.BlockSpec((_B, _NCLS), lambda i: (0, 0)),
        ],
        out_shape=[jax.ShapeDtypeStruct((_B, _C, _C), f32),
                   jax.ShapeDtypeStruct((_B, _NCLS), f32)],
    )(t0[0, :_N], t0[1, :_N], t1[0, :_N], t1[1, :_N], snb,
      sph_sum, snb_sum, xp,
      W_ac_self, W_ac_nb, b_ac.reshape(1, _C),
      W_lin, b_lin.reshape(1, _NCLS))

    return logits
